# baseline (speedup 1.0000x reference)
import jax
import jax.numpy as jnp
from jax.experimental import pallas as pl
from jax.experimental.pallas import tpu as pltpu

SIZE = 8
S = SIZE * SIZE
P = 448 // SIZE
NEG = -jnp.inf


def _rank_body(rcm0_ref, dist_ref, acc_ref, srcb_ref, rcm_ref, er_ref, ec_ref):
    rcm_ref[:] = rcm0_ref[:]
    er_ref[:] = jnp.zeros_like(er_ref)
    ec_ref[:] = jnp.zeros_like(ec_ref)
    srcb_ref[:] = jnp.zeros_like(srcb_ref)

    riota = jax.lax.broadcasted_iota(jnp.int32, (1, S, S), 1)
    ciota = jax.lax.broadcasted_iota(jnp.int32, (1, S, S), 2)
    riota_r = jax.lax.broadcasted_iota(jnp.int32, (1, S, 1), 1)
    ciota_c = jax.lax.broadcasted_iota(jnp.int32, (1, 1, S), 2)
    fiota = riota * S + ciota
    diota = jax.lax.broadcasted_iota(jnp.int32, (1, S), 1)

    def body(i, _):
        a = rcm_ref[:] + dist_ref[i][None, :, :]
        rcm_ref[:] = a
        w = a + er_ref[:] + ec_ref[:]
        m1 = jnp.max(w, axis=(1, 2), keepdims=True)
        eq = w == m1
        fi = jnp.min(jnp.where(eq, fiota, S * S), axis=(1, 2), keepdims=True)
        rsel = fi // S
        csel = fi - rsel * S
        er_ref[:] = er_ref[:] + jnp.where(riota_r == rsel, NEG, 0.0)
        ec_ref[:] = ec_ref[:] + jnp.where(ciota_c == csel, NEG, 0.0)
        r2 = jnp.squeeze(rsel, -1)
        c2 = jnp.squeeze(csel, -1)
        srcb_ref[:] = srcb_ref[:] + jnp.where(diota == c2, r2, 0)
        return 0

    jax.lax.fori_loop(0, S, body, 0)
    acc_ref[:] = jnp.where(srcb_ref[:][:, None, :] == riota, 1.0, 0.0)


def _gather_body(t_ref, x_ref, o_ref, blk_ref):
    b = pl.program_id(0)
    B = pl.num_programs(0)
    for sh in range(SIZE):
        slab = x_ref[0, :, sh * P:(sh + 1) * P, :]
        for sw in range(SIZE):
            blk_ref[sh * SIZE + sw] = slab[:, :, sw * P:(sw + 1) * P]
    for h in range(SIZE):
        cols = []
        for w in range(SIZE):
            k = t_ref[B - 1 - b, h * SIZE + w]
            cols.append(blk_ref[k])
        o_ref[0, :, h * P:(h + 1) * P, :] = jnp.concatenate(cols, axis=-1)


def _make_disturb():
    keys = jax.random.split(jax.random.key(42), S)
    return jax.vmap(lambda k: jax.random.normal(k, (S, S), dtype=jnp.float32))(
        keys
    ) * 1e-6


def kernel(x, params, batch_seq, batchsize):
    B, C = x.shape[0], x.shape[1]
    start = batch_seq * batchsize
    sel = jax.lax.dynamic_slice_in_dim(params, start, B, axis=0)
    rcm0 = jax.nn.softmax(sel, axis=1)
    disturb = _make_disturb()

    perm, srcb = pl.pallas_call(
        _rank_body,
        out_shape=(
            jax.ShapeDtypeStruct((B, S, S), jnp.float32),
            jax.ShapeDtypeStruct((B, S), jnp.int32),
        ),
        scratch_shapes=[
            pltpu.VMEM((B, S, S), jnp.float32),
            pltpu.VMEM((B, S, 1), jnp.float32),
            pltpu.VMEM((B, 1, S), jnp.float32),
        ],
    )(rcm0, disturb)

    out = pl.pallas_call(
        _gather_body,
        grid_spec=pltpu.PrefetchScalarGridSpec(
            num_scalar_prefetch=1,
            grid=(B,),
            in_specs=[pl.BlockSpec((1, C, 448, 448), lambda b, f: (b, 0, 0, 0))],
            out_specs=pl.BlockSpec((1, C, 448, 448), lambda b, f: (b, 0, 0, 0)),
            scratch_shapes=[pltpu.VMEM((S, C, P, P), jnp.float32)],
        ),
        out_shape=jax.ShapeDtypeStruct((B, C, 448, 448), jnp.float32),
    )(srcb, x)

    result_rev = perm[::-1]
    result = jnp.stack([result_rev, result_rev, result_rev], axis=1)
    return out, result


# speedup vs baseline: 1.0823x; 1.0823x over previous
import jax
import jax.numpy as jnp
from jax.experimental import pallas as pl
from jax.experimental.pallas import tpu as pltpu

SIZE = 8
S = SIZE * SIZE
P = 448 // SIZE
NEG = -jnp.inf


def _rank_body(rcm0_ref, dist_ref, acc_ref, srcb_ref, rcm_ref, msk_ref):
    rcm_ref[:] = rcm0_ref[:]
    msk_ref[:] = jnp.zeros_like(msk_ref)
    srcb_ref[:] = jnp.zeros_like(srcb_ref)

    riota = jax.lax.broadcasted_iota(jnp.int32, (1, S, S), 1)
    ciota = jax.lax.broadcasted_iota(jnp.int32, (1, S, S), 2)
    fiota = riota * S + ciota
    diota = jax.lax.broadcasted_iota(jnp.int32, (1, S), 1)

    def body(i, _):
        a = rcm_ref[:] + dist_ref[i][None, :, :]
        rcm_ref[:] = a
        w = a + msk_ref[:]
        m1 = jnp.max(w, axis=(1, 2), keepdims=True)
        eq = w == m1
        fi = jnp.min(jnp.where(eq, fiota, S * S), axis=(1, 2), keepdims=True)
        rsel = fi // S
        csel = fi - rsel * S
        hit = (riota == rsel) | (ciota == csel)
        msk_ref[:] = msk_ref[:] + jnp.where(hit, NEG, 0.0)
        r2 = jnp.squeeze(rsel, -1)
        c2 = jnp.squeeze(csel, -1)
        srcb_ref[:] = srcb_ref[:] + jnp.where(diota == c2, r2, 0)
        return 0

    jax.lax.fori_loop(0, S, body, 0)
    acc_ref[:] = jnp.where(srcb_ref[:][:, None, :] == riota, 1.0, 0.0)


def _gather_body(t_ref, x_ref, o_ref, blk_ref):
    b = pl.program_id(0)
    B = pl.num_programs(0)
    for sh in range(SIZE):
        slab = x_ref[0, :, sh * P:(sh + 1) * P, :]
        for sw in range(SIZE):
            blk_ref[sh * SIZE + sw] = slab[:, :, sw * P:(sw + 1) * P]
    for h in range(SIZE):
        cols = []
        for w in range(SIZE):
            k = t_ref[B - 1 - b, h * SIZE + w]
            cols.append(blk_ref[k])
        o_ref[0, :, h * P:(h + 1) * P, :] = jnp.concatenate(cols, axis=-1)


def _make_disturb():
    keys = jax.random.split(jax.random.key(42), S)
    return jax.vmap(lambda k: jax.random.normal(k, (S, S), dtype=jnp.float32))(
        keys
    ) * 1e-6


def kernel(x, params, batch_seq, batchsize):
    B, C = x.shape[0], x.shape[1]
    start = batch_seq * batchsize
    sel = jax.lax.dynamic_slice_in_dim(params, start, B, axis=0)
    rcm0 = jax.nn.softmax(sel, axis=1)
    disturb = _make_disturb()

    perm, srcb = pl.pallas_call(
        _rank_body,
        out_shape=(
            jax.ShapeDtypeStruct((B, S, S), jnp.float32),
            jax.ShapeDtypeStruct((B, S), jnp.int32),
        ),
        scratch_shapes=[
            pltpu.VMEM((B, S, S), jnp.float32),
            pltpu.VMEM((B, S, S), jnp.float32),
        ],
    )(rcm0, disturb)

    out = pl.pallas_call(
        _gather_body,
        grid_spec=pltpu.PrefetchScalarGridSpec(
            num_scalar_prefetch=1,
            grid=(B,),
            in_specs=[pl.BlockSpec((1, C, 448, 448), lambda b, f: (b, 0, 0, 0))],
            out_specs=pl.BlockSpec((1, C, 448, 448), lambda b, f: (b, 0, 0, 0)),
            scratch_shapes=[pltpu.VMEM((S, C, P, P), jnp.float32)],
        ),
        out_shape=jax.ShapeDtypeStruct((B, C, 448, 448), jnp.float32),
    )(srcb, x)

    result_rev = perm[::-1]
    result = jnp.stack([result_rev, result_rev, result_rev], axis=1)
    return out, result


# speedup vs baseline: 1.2136x; 1.1213x over previous
import jax
import jax.numpy as jnp
from jax.experimental import pallas as pl
from jax.experimental.pallas import tpu as pltpu

SIZE = 8
S = SIZE * SIZE
P = 448 // SIZE
NEG = -jnp.inf


def _rank_body(rcm0_ref, dist_ref, acc_ref, srcb_ref, rcm_ref, msk_ref):
    rcm_ref[:] = rcm0_ref[:]
    msk_ref[:] = jnp.zeros_like(msk_ref)
    srcb_ref[:] = jnp.zeros_like(srcb_ref)

    fiota = jax.lax.broadcasted_iota(jnp.int32, (1, S * S), 1)
    riota = fiota // S
    ciota = fiota - riota * S
    diota = jax.lax.broadcasted_iota(jnp.int32, (1, S), 1)

    def body(i, _):
        a = rcm_ref[:] + dist_ref[i][None, :]
        rcm_ref[:] = a
        w = a + msk_ref[:]
        m1 = jnp.max(w, axis=1, keepdims=True)
        eq = w == m1
        fi = jnp.min(jnp.where(eq, fiota, S * S), axis=1, keepdims=True)
        rsel = fi // S
        csel = fi - rsel * S
        hit = (riota == rsel) | (ciota == csel)
        msk_ref[:] = msk_ref[:] + jnp.where(hit, NEG, 0.0)
        srcb_ref[:] = srcb_ref[:] + jnp.where(diota == csel, rsel, 0)
        return 0

    jax.lax.fori_loop(0, S, body, 0)
    riota3 = jax.lax.broadcasted_iota(jnp.int32, (1, S, S), 1)
    acc_ref[:] = jnp.where(srcb_ref[:][:, None, :] == riota3, 1.0, 0.0)


def _gather_body(t_ref, x_ref, o_ref, blk_ref):
    b = pl.program_id(0)
    B = pl.num_programs(0)
    for sh in range(SIZE):
        slab = x_ref[0, :, sh * P:(sh + 1) * P, :]
        for sw in range(SIZE):
            blk_ref[sh * SIZE + sw] = slab[:, :, sw * P:(sw + 1) * P]
    for h in range(SIZE):
        cols = []
        for w in range(SIZE):
            k = t_ref[B - 1 - b, h * SIZE + w]
            cols.append(blk_ref[k])
        o_ref[0, :, h * P:(h + 1) * P, :] = jnp.concatenate(cols, axis=-1)


def _make_disturb():
    keys = jax.random.split(jax.random.key(42), S)
    return jax.vmap(lambda k: jax.random.normal(k, (S, S), dtype=jnp.float32))(
        keys
    ) * 1e-6


def kernel(x, params, batch_seq, batchsize):
    B, C = x.shape[0], x.shape[1]
    start = batch_seq * batchsize
    sel = jax.lax.dynamic_slice_in_dim(params, start, B, axis=0)
    rcm0 = jax.nn.softmax(sel, axis=1)
    disturb = _make_disturb()

    perm, srcb = pl.pallas_call(
        _rank_body,
        out_shape=(
            jax.ShapeDtypeStruct((B, S, S), jnp.float32),
            jax.ShapeDtypeStruct((B, S), jnp.int32),
        ),
        scratch_shapes=[
            pltpu.VMEM((B, S * S), jnp.float32),
            pltpu.VMEM((B, S * S), jnp.float32),
        ],
    )(rcm0.reshape(B, S * S), disturb.reshape(S, S * S))

    out = pl.pallas_call(
        _gather_body,
        grid_spec=pltpu.PrefetchScalarGridSpec(
            num_scalar_prefetch=1,
            grid=(B,),
            in_specs=[pl.BlockSpec((1, C, 448, 448), lambda b, f: (b, 0, 0, 0))],
            out_specs=pl.BlockSpec((1, C, 448, 448), lambda b, f: (b, 0, 0, 0)),
            scratch_shapes=[pltpu.VMEM((S, C, P, P), jnp.float32)],
        ),
        out_shape=jax.ShapeDtypeStruct((B, C, 448, 448), jnp.float32),
    )(srcb, x)

    result_rev = perm[::-1]
    result = jnp.stack([result_rev, result_rev, result_rev], axis=1)
    return out, result
